# Initial kernel scaffold; baseline (speedup 1.0000x reference)
import jax
import jax.numpy as jnp
from jax.experimental import pallas as pl


def kernel(x, edge_index, edge_attr, edge_types, type_emb_gat, W_gat, W_edge_gat, att_src, att_dst, att_edge, bias_gat, type_emb_gine, edge_lin_w, edge_lin_b, mlp_w1, mlp_b1, mlp_w2, mlp_b2, comb_w, comb_b, ln_gamma, ln_beta):
    raise NotImplementedError("write your pallas kernel here")


# speedup vs baseline: 19.9957x; 19.9957x over previous
import functools
import jax
import jax.numpy as jnp
from jax import lax
from jax.experimental import pallas as pl
from jax.experimental.pallas import tpu as pltpu
from jax.experimental.pallas import tpu_sc as plsc

N = 10000
E = 160000
D = 128
ED = 16
H = 4
C = 64
T = 8
GINE = 64
OUT = 128

NB = 1000
EB = 2000
ZW = D + GINE

BW = E // 32
BK = 1000
BG = 125
CT = E // 16
CK = 250
DK = 500
CG = 125

def _get_mesh():
    return plsc.VectorSubcoreMesh(core_axis_name="c", subcore_axis_name="s")


def _nodes_tc(x_ref, wg_ref, wa_ref, h_ref, xs_ref, asd_ref):
    xb = x_ref[...]
    r = jnp.dot(xb, wg_ref[...], preferred_element_type=jnp.float32)
    h_ref[0] = r[:, :D]
    h_ref[1] = r[:, D:]
    xs_ref[0] = xb[:, :GINE]
    xs_ref[1] = xb[:, GINE:]
    a = jnp.dot(xb, wa_ref[...], preferred_element_type=jnp.float32)
    asd_ref[...] = jnp.concatenate([a, jnp.zeros((NB, 2 * H), jnp.float32)], axis=1)


def _edges_tc(ea_ref, et_ref, wae_ref, wep_ref, elb_ref, ae_ref, ep_ref):
    tvals = jnp.arange(T, dtype=jnp.int32)[None, :].astype(jnp.float32)
    oh = (et_ref[...] == tvals).astype(jnp.float32)
    ea = jnp.concatenate([ea_ref[...], oh], axis=1)
    ae_ref[...] = jnp.dot(ea, wae_ref[...], preferred_element_type=jnp.float32)
    ep = jnp.dot(ea, wep_ref[...], preferred_element_type=jnp.float32) + elb_ref[...]
    ep_ref[0] = ep[:, :GINE]
    ep_ref[1] = ep[:, GINE:]


def _attn_sc(srcg_h, dstg_h, dst_h, asd_h, ae_h, ri_h,
             ex_h, dp_h,
             sidx, didx, dstv, sbuf, dbuf, aev, exv, dnv, idxm, acc, sem):
    c = lax.axis_index("c")
    s = lax.axis_index("s")
    w = c * 16 + s
    lane = jnp.arange(16, dtype=jnp.int32)
    e8l = lane & 7
    hl = lane >> 3

    zv = jnp.zeros((16,), jnp.float32)

    def zinit(i, cy):
        dnv[i, pl.ds(0, 16)] = zv
        return cy

    lax.fori_loop(0, (N * H) // 16, zinit, 0)
    pltpu.sync_copy(ri_h, idxm)

    @pl.when(s == 0)
    def _():
        pltpu.sync_copy(dnv, acc)

    plsc.subcore_barrier()

    def chunk(j, carry):
        base = w * BW + j * BK
        rowb = w * (BW // BG) + j * (BK // BG)
        pltpu.sync_copy(srcg_h.at[pl.ds(rowb, BK // BG)], sidx)
        pltpu.sync_copy(dstg_h.at[pl.ds(rowb, BK // BG)], didx)
        gs = [pltpu.async_copy(asd_h.at[sidx.at[g]],
                               sbuf.at[pl.ds(g * BG, BG)], sem)
              for g in range(BK // BG)]
        gs += [pltpu.async_copy(asd_h.at[didx.at[g]],
                                dbuf.at[pl.ds(g * BG, BG)], sem)
               for g in range(BK // BG)]
        pltpu.sync_copy(dst_h.at[pl.ds(base, BK)], dstv)
        pltpu.sync_copy(ae_h.at[pl.ds(base, BK)], aev)
        for d in gs:
            d.wait()

        def vec(i, cy):
            e8 = i * 8 + e8l
            dstrep = plsc.load_gather(dstv, [e8])
            for hp in (0, 2):
                hvec = hp + hl
                vs = plsc.load_gather(sbuf, [e8, hvec])
                vd = plsc.load_gather(dbuf, [e8, hvec + 4])
                ve = plsc.load_gather(aev, [e8, hvec])
                l = vs + vd + ve
                l = jnp.maximum(l, 0.2 * l)
                xv = jnp.exp(l)
                plsc.store_scatter(exv, [e8, hvec], xv)
                fl = dstrep * 4 + hvec
                plsc.addupdate_scatter(dnv, [fl >> 4, fl & 15], xv)
            return cy

        lax.fori_loop(0, BK // 8, vec, 0)
        pltpu.sync_copy(exv, ex_h.at[pl.ds(base, BK)])
        return carry

    lax.fori_loop(0, BW // BK, chunk, 0)
    ms = [pltpu.async_copy(dnv.at[pl.ds(g * BG, BG)], acc.at[idxm.at[g]],
                           sem, add=True) for g in range((N * H // 16) // BG)]
    for d in ms:
        d.wait()
    plsc.subcore_barrier()

    @pl.when(s == 0)
    def _():
        pltpu.sync_copy(acc, dp_h.at[c])


def _gat_sc(srcs_h, dstg_h, h_h, ex_h, zz_h,
            numer_h,
            idxs, idxd, buf, exv, acc, sem, sem2):
    c = lax.axis_index("c")
    s = lax.axis_index("s")
    twoc = 2 * c

    pltpu.sync_copy(zz_h, acc.at[pl.ds(s * (N // 16), N // 16)])
    plsc.subcore_barrier()

    def chunk(j, carry):
        base = s * CT + j * CK
        rowb = s * (CT // CG) + j * (CK // CG)
        pltpu.sync_copy(srcs_h.at[c, pl.ds(rowb, CK // CG)], idxs)
        pltpu.sync_copy(dstg_h.at[pl.ds(rowb, CK // CG)], idxd)
        gs = [pltpu.async_copy(h_h.at[idxs.at[g]], buf.at[pl.ds(g * CG, CG)], sem)
              for g in range(CK // CG)]
        pltpu.sync_copy(ex_h.at[pl.ds(base * H, CK * H)], exv)
        for d in gs:
            d.wait()

        def edge(e, cy):
            i0 = jnp.full((16,), 4 * e, dtype=jnp.int32) + twoc
            a0 = plsc.load_gather(exv, [i0])
            a1 = plsc.load_gather(exv, [i0 + 1])
            for q in range(4):
                sl = pl.ds(q * 16, 16)
                buf[e, sl] = buf[e, sl] * a0
            for q in range(4):
                sl = pl.ds(64 + q * 16, 16)
                buf[e, sl] = buf[e, sl] * a1
            return cy

        lax.fori_loop(0, CK, edge, 0)
        ss = [pltpu.async_copy(buf.at[pl.ds(g * CG, CG)], acc.at[idxd.at[g]],
                               sem2, add=True) for g in range(CK // CG)]
        for d in ss:
            d.wait()
        return carry

    lax.fori_loop(0, CT // CK, chunk, 0)
    plsc.subcore_barrier()

    @pl.when(s == 0)
    def _():
        pltpu.sync_copy(acc, numer_h.at[c])


def _gine_sc(srcs_h, dstg_h, xs_h, ep_h, zz_h,
             agg_h,
             idxs, idxd, buf, epv, acc, sem, sem2):
    c = lax.axis_index("c")
    s = lax.axis_index("s")

    pltpu.sync_copy(zz_h, acc.at[pl.ds(s * (N // 16), N // 16)])
    plsc.subcore_barrier()

    def chunk(j, carry):
        base = s * CT + j * DK
        rowb = s * (CT // CG) + j * (DK // CG)
        pltpu.sync_copy(srcs_h.at[c, pl.ds(rowb, DK // CG)], idxs)
        pltpu.sync_copy(dstg_h.at[pl.ds(rowb, DK // CG)], idxd)
        gs = [pltpu.async_copy(xs_h.at[idxs.at[g]], buf.at[pl.ds(g * CG, CG)], sem)
              for g in range(DK // CG)]
        pltpu.sync_copy(ep_h.at[c, pl.ds(base, DK)], epv)
        for d in gs:
            d.wait()

        def edge(e, cy):
            for q in range(4):
                sl = pl.ds(q * 16, 16)
                buf[e, sl] = jnp.maximum(buf[e, sl] + epv[e, sl], 0.0)
            return cy

        lax.fori_loop(0, DK, edge, 0)
        ss = [pltpu.async_copy(buf.at[pl.ds(g * CG, CG)], acc.at[idxd.at[g]],
                               sem2, add=True) for g in range(DK // CG)]
        for d in ss:
            d.wait()
        return carry

    lax.fori_loop(0, CT // DK, chunk, 0)
    plsc.subcore_barrier()

    @pl.when(s == 0)
    def _():
        pltpu.sync_copy(acc, agg_h.at[c])


def _final_tc(num_ref, dp_ref, agg_ref, x_ref, w1_ref, b1_ref, w2_ref, b2_ref,
              cwg0_ref, cwg1_ref, cwi_ref, cb_ref, lng_ref, lnb_ref, o_ref):
    d = dp_ref[0] + dp_ref[1] + 1e-16
    dd0 = jnp.concatenate([jnp.broadcast_to(d[:, 0:1], (NB, C)),
                           jnp.broadcast_to(d[:, 1:2], (NB, C))], axis=1)
    dd1 = jnp.concatenate([jnp.broadcast_to(d[:, 2:3], (NB, C)),
                           jnp.broadcast_to(d[:, 3:4], (NB, C))], axis=1)
    g0 = num_ref[0] / dd0
    g1 = num_ref[1] / dd1
    hg = x_ref[...] + jnp.concatenate([agg_ref[0], agg_ref[1]], axis=1)
    m1 = jnp.maximum(jnp.dot(hg, w1_ref[...], preferred_element_type=jnp.float32)
                     + b1_ref[...], 0.0)
    gi = jnp.dot(m1, w2_ref[...], preferred_element_type=jnp.float32) + b2_ref[...]
    z = (jnp.dot(g0, cwg0_ref[...], preferred_element_type=jnp.float32)
         + jnp.dot(g1, cwg1_ref[...], preferred_element_type=jnp.float32)
         + jnp.dot(gi, cwi_ref[...], preferred_element_type=jnp.float32)
         + cb_ref[...])
    mu = jnp.mean(z, axis=1, keepdims=True)
    zc = z - mu
    var = jnp.mean(zc * zc, axis=1, keepdims=True)
    zn = zc * jax.lax.rsqrt(var + 1e-5) * lng_ref[...] + lnb_ref[...]
    o_ref[...] = jnp.maximum(zn, 0.0)


def kernel(x, edge_index, edge_attr, edge_types, type_emb_gat, W_gat, W_edge_gat,
           att_src, att_dst, att_edge, bias_gat, type_emb_gine, edge_lin_w,
           edge_lin_b, mlp_w1, mlp_b1, mlp_w2, mlp_b2, comb_w, comb_b,
           ln_gamma, ln_beta):
    f32 = jnp.float32
    src = edge_index[0].astype(jnp.int32)
    dst = edge_index[1].astype(jnp.int32)
    et = edge_types.astype(jnp.int32)

    wg3 = W_gat.reshape(D, H, C)
    we3 = W_edge_gat.reshape(ED, H, C)
    Was = jnp.einsum('dhc,hc->dh', wg3, att_src)
    Wad = jnp.einsum('dhc,hc->dh', wg3, att_dst)
    Wae = jnp.einsum('dhc,hc->dh', we3, att_edge)
    WA = jnp.concatenate([Was, Wad], axis=1)
    tA = type_emb_gat @ Wae
    tG = type_emb_gine @ edge_lin_w
    Wae_ext = jnp.concatenate([Wae, tA], axis=0)
    Wep_ext = jnp.concatenate([edge_lin_w, tG], axis=0)
    cb_eff = (comb_b + bias_gat @ comb_w[:H * C]).reshape(1, OUT)

    etf = et.astype(f32).reshape(E, 1)

    srcs = jnp.stack([src, src + N]).reshape(2, E // CG, CG)
    srcg125 = src.reshape(E // BG, BG)
    dstg125 = dst.reshape(E // BG, BG)
    rowiota = jnp.arange(N * H // 16, dtype=jnp.int32).reshape(-1, BG)

    z128 = jnp.zeros((N // 16, D), f32)
    z64 = jnp.zeros((N // 16, GINE), f32)

    h2, xs, asd = pl.pallas_call(
        _nodes_tc,
        grid=(N // NB,),
        in_specs=[
            pl.BlockSpec((NB, D), lambda i: (i, 0)),
            pl.BlockSpec((D, H * C), lambda i: (0, 0)),
            pl.BlockSpec((D, 2 * H), lambda i: (0, 0)),
        ],
        out_specs=[
            pl.BlockSpec((2, NB, D), lambda i: (0, i, 0)),
            pl.BlockSpec((2, NB, GINE), lambda i: (0, i, 0)),
            pl.BlockSpec((NB, 4 * H), lambda i: (i, 0)),
        ],
        out_shape=[
            jax.ShapeDtypeStruct((2, N, D), f32),
            jax.ShapeDtypeStruct((2, N, GINE), f32),
            jax.ShapeDtypeStruct((N, 4 * H), f32),
        ],
    )(x, W_gat, WA)
    h2f = h2.reshape(2 * N, D)
    xsf = xs.reshape(2 * N, GINE)

    ae, ep2 = pl.pallas_call(
        _edges_tc,
        grid=(E // EB,),
        in_specs=[
            pl.BlockSpec((EB, ED), lambda i: (i, 0)),
            pl.BlockSpec((EB, 1), lambda i: (i, 0)),
            pl.BlockSpec((ED + T, H), lambda i: (0, 0)),
            pl.BlockSpec((ED + T, D), lambda i: (0, 0)),
            pl.BlockSpec((1, D), lambda i: (0, 0)),
        ],
        out_specs=[
            pl.BlockSpec((EB, H), lambda i: (i, 0)),
            pl.BlockSpec((2, EB, GINE), lambda i: (0, i, 0)),
        ],
        out_shape=[
            jax.ShapeDtypeStruct((E, H), f32),
            jax.ShapeDtypeStruct((2, E, GINE), f32),
        ],
    )(edge_attr, etf, Wae_ext, Wep_ext, edge_lin_b.reshape(1, D))

    ex, dp = pl.kernel(
        _attn_sc,
        out_type=[
            jax.ShapeDtypeStruct((E, H), f32),
            jax.ShapeDtypeStruct((2, N * H // 16, 16), f32),
        ],
        mesh=_get_mesh(),
        compiler_params=pltpu.CompilerParams(needs_layout_passes=False, use_tc_tiling_on_sc=False),
        scratch_types=[
            pltpu.VMEM((BK // BG, BG), jnp.int32),
            pltpu.VMEM((BK // BG, BG), jnp.int32),
            pltpu.VMEM((BK,), jnp.int32),
            pltpu.VMEM((BK, 4 * H), f32),
            pltpu.VMEM((BK, 4 * H), f32),
            pltpu.VMEM((BK, H), f32),
            pltpu.VMEM((BK, H), f32),
            pltpu.VMEM((N * H // 16, 16), f32),
            pltpu.VMEM(((N * H // 16) // BG, BG), jnp.int32),
            pltpu.VMEM_SHARED((N * H // 16, 16), f32),
            pltpu.SemaphoreType.DMA,
        ],
    )(srcg125, dstg125, dst, asd, ae, rowiota)
    dp = dp.reshape(2, N, H)

    numer = pl.kernel(
        _gat_sc,
        out_type=jax.ShapeDtypeStruct((2, N, D), f32),
        mesh=_get_mesh(),
        compiler_params=pltpu.CompilerParams(needs_layout_passes=False, use_tc_tiling_on_sc=False),
        scratch_types=[
            pltpu.VMEM((CK // CG, CG), jnp.int32),
            pltpu.VMEM((CK // CG, CG), jnp.int32),
            pltpu.VMEM((CK, D), f32),
            pltpu.VMEM((CK * H,), f32),
            pltpu.VMEM_SHARED((N, D), f32),
            pltpu.SemaphoreType.DMA,
            pltpu.SemaphoreType.DMA,
        ],
    )(srcs, dstg125, h2f, ex.reshape(E * H), z128)

    agg = pl.kernel(
        _gine_sc,
        out_type=jax.ShapeDtypeStruct((2, N, GINE), f32),
        mesh=_get_mesh(),
        compiler_params=pltpu.CompilerParams(needs_layout_passes=False, use_tc_tiling_on_sc=False),
        scratch_types=[
            pltpu.VMEM((DK // CG, CG), jnp.int32),
            pltpu.VMEM((DK // CG, CG), jnp.int32),
            pltpu.VMEM((DK, GINE), f32),
            pltpu.VMEM((DK, GINE), f32),
            pltpu.VMEM_SHARED((N, GINE), f32),
            pltpu.SemaphoreType.DMA,
            pltpu.SemaphoreType.DMA,
        ],
    )(srcs, dstg125, xsf, ep2, z64)

    out = pl.pallas_call(
        _final_tc,
        grid=(N // NB,),
        in_specs=[
            pl.BlockSpec((2, NB, D), lambda i: (0, i, 0)),
            pl.BlockSpec((2, NB, H), lambda i: (0, i, 0)),
            pl.BlockSpec((2, NB, GINE), lambda i: (0, i, 0)),
            pl.BlockSpec((NB, D), lambda i: (i, 0)),
            pl.BlockSpec((D, GINE), lambda i: (0, 0)),
            pl.BlockSpec((1, GINE), lambda i: (0, 0)),
            pl.BlockSpec((GINE, GINE), lambda i: (0, 0)),
            pl.BlockSpec((1, GINE), lambda i: (0, 0)),
            pl.BlockSpec((D, OUT), lambda i: (0, 0)),
            pl.BlockSpec((D, OUT), lambda i: (0, 0)),
            pl.BlockSpec((GINE, OUT), lambda i: (0, 0)),
            pl.BlockSpec((1, OUT), lambda i: (0, 0)),
            pl.BlockSpec((1, OUT), lambda i: (0, 0)),
            pl.BlockSpec((1, OUT), lambda i: (0, 0)),
        ],
        out_specs=pl.BlockSpec((NB, OUT), lambda i: (i, 0)),
        out_shape=jax.ShapeDtypeStruct((N, OUT), f32),
    )(numer, dp, agg, x, mlp_w1, mlp_b1.reshape(1, GINE), mlp_w2,
      mlp_b2.reshape(1, GINE), comb_w[:D], comb_w[D:2 * D], comb_w[2 * D:],
      cb_eff, ln_gamma.reshape(1, OUT), ln_beta.reshape(1, OUT))
    return out
